# baseline (speedup 1.0000x reference)
import functools

import jax
import jax.numpy as jnp
from jax import lax
from jax.experimental import pallas as pl
from jax.experimental.pallas import tpu as pltpu
from jax.experimental.pallas import tpu_sc as plsc

_N = 10000
_E = 320000
_NPAD = 10240
_NW = 32
_K = 96
_EPW = 10368
_EPAD = _NW * _EPW
_NCH = _EPW // _K
_RPT = _NPAD // 16
_BLK = 1280
_G = _NPAD // _BLK


def _prep1_body(x_ref, w1_ref, as_ref, ad_ref, t1_ref, t2_ref):
    h = jnp.dot(x_ref[...], w1_ref[...], preferred_element_type=jnp.float32)
    asrc = lax.dot_general(h, as_ref[...], (((1,), (0,)), ((), ())))
    adst = lax.dot_general(h, ad_ref[...], (((1,), (0,)), ((), ())))
    z8 = jnp.zeros_like(asrc)
    t1_ref[...] = jnp.concatenate([h, asrc, z8], axis=1)
    t2_ref[...] = jnp.concatenate([adst, z8], axis=1)


def _prep1(xp, W1, As1, Ad1):
    return pl.pallas_call(
        _prep1_body,
        grid=(_G,),
        in_specs=[
            pl.BlockSpec((_BLK, 128), lambda i: (i, 0)),
            pl.BlockSpec((128, 128), lambda i: (0, 0)),
            pl.BlockSpec((128, 8), lambda i: (0, 0)),
            pl.BlockSpec((128, 8), lambda i: (0, 0)),
        ],
        out_specs=[
            pl.BlockSpec((_BLK, 144), lambda i: (i, 0)),
            pl.BlockSpec((_BLK, 16), lambda i: (i, 0)),
        ],
        out_shape=[
            jax.ShapeDtypeStruct((_NPAD, 144), jnp.float32),
            jax.ShapeDtypeStruct((_NPAD, 16), jnp.float32),
        ],
    )(xp, W1, As1, Ad1)


def _prep2_body(acc_ref, b1_ref, w2_ref, as2_ref, ad2_ref, r_ref, t1_ref, t2_ref):
    s = acc_ref[0] + acc_ref[1]
    den = s[:, 128:136]
    den = jnp.where(den != 0.0, den, 1.0)
    inv128 = lax.dot_general(1.0 / den, r_ref[...], (((1,), (0,)), ((), ())))
    h1 = s[:, 0:128] * inv128 + b1_ref[...]
    neg = jnp.exp(jnp.minimum(h1, 0.0)) - 1.0
    h1 = jnp.where(h1 > 0.0, h1, neg)
    h2 = jnp.dot(h1, w2_ref[...], preferred_element_type=jnp.float32)
    a2s = lax.dot_general(h2, as2_ref[...], (((1,), (0,)), ((), ())))
    a2d = lax.dot_general(h2, ad2_ref[...], (((1,), (0,)), ((), ())))
    z15 = jnp.zeros((h2.shape[0], 15), jnp.float32)
    t1_ref[...] = jnp.concatenate([h2, a2s, z15], axis=1)
    t2_ref[...] = jnp.concatenate([a2d, z15], axis=1)


def _prep2(acc1, b1, W2, As2, Ad2, R):
    return pl.pallas_call(
        _prep2_body,
        grid=(_G,),
        in_specs=[
            pl.BlockSpec((2, _BLK, 144), lambda i: (0, i, 0)),
            pl.BlockSpec((1, 128), lambda i: (0, 0)),
            pl.BlockSpec((128, 64), lambda i: (0, 0)),
            pl.BlockSpec((64, 1), lambda i: (0, 0)),
            pl.BlockSpec((64, 1), lambda i: (0, 0)),
            pl.BlockSpec((8, 128), lambda i: (0, 0)),
        ],
        out_specs=[
            pl.BlockSpec((_BLK, 80), lambda i: (i, 0)),
            pl.BlockSpec((_BLK, 16), lambda i: (i, 0)),
        ],
        out_shape=[
            jax.ShapeDtypeStruct((_NPAD, 80), jnp.float32),
            jax.ShapeDtypeStruct((_NPAD, 16), jnp.float32),
        ],
    )(acc1, b1, W2, As2, Ad2, R)


def _final_body(acc_ref, b2_ref, o_ref):
    s = acc_ref[0] + acc_ref[1]
    den = s[:, 64:65]
    den = jnp.where(den != 0.0, den, 1.0)
    o = s[:, 0:64] / den + b2_ref[...]
    m = jnp.max(o, axis=1, keepdims=True)
    z = o - m
    lse = jnp.log(jnp.sum(jnp.exp(z), axis=1, keepdims=True))
    o_ref[...] = z - lse


def _final(acc2, b2):
    return pl.pallas_call(
        _final_body,
        grid=(_G,),
        in_specs=[
            pl.BlockSpec((2, _BLK, 80), lambda i: (0, i, 0)),
            pl.BlockSpec((1, 64), lambda i: (0, 0)),
        ],
        out_specs=pl.BlockSpec((_BLK, 64), lambda i: (i, 0)),
        out_shape=jax.ShapeDtypeStruct((_NPAD, 64), jnp.float32),
    )(acc2, b2)


def _make_edge_kernel(pw, acol, heads, csz):
    hc = acol
    mesh = plsc.VectorSubcoreMesh(core_axis_name="c", subcore_axis_name="s")

    @functools.partial(
        pl.kernel,
        mesh=mesh,
        compiler_params=pltpu.CompilerParams(
            use_tc_tiling_on_sc=False, needs_layout_passes=False),
        out_type=jax.ShapeDtypeStruct((2, _NPAD, pw), jnp.float32),
        scratch_types=[
            pltpu.VMEM((2, _K), jnp.int32),
            pltpu.VMEM((2, _K), jnp.int32),
            pltpu.VMEM((_K, pw), jnp.float32),
            pltpu.VMEM((_K, pw), jnp.float32),
            pltpu.VMEM((_K, 16), jnp.float32),
            pltpu.VMEM((_K, 16), jnp.float32),
            pltpu.VMEM((_K, 16), jnp.float32),
            pltpu.VMEM((16, 16), jnp.float32),
            pltpu.VMEM_SHARED((_NPAD, pw), jnp.float32),
            pltpu.SemaphoreType.DMA,
            pltpu.SemaphoreType.DMA,
            pltpu.SemaphoreType.DMA,
            pltpu.SemaphoreType.DMA,
            pltpu.SemaphoreType.DMA,
            pltpu.SemaphoreType.DMA,
        ],
    )
    def ek(t1_hbm, t2_hbm, sdi_hbm, lb_hbm, zer_hbm, out_hbm,
           sd0, sd1, rows0, rows1, t2r0, t2r1, wb, lbv, acc,
           sg10, sg11, sg20, sg21, ssc0, ssc1):
        cid = lax.axis_index("c")
        sid = lax.axis_index("s")
        gwid = cid * 16 + sid

        pltpu.sync_copy(zer_hbm.at[pl.ds(sid * _RPT, _RPT)],
                        acc.at[pl.ds(sid * _RPT, _RPT)])
        pltpu.sync_copy(lb_hbm, lbv)
        plsc.subcore_barrier()

        cbase = gwid * jnp.int32(_NCH)
        bufs = ((sd0, rows0, t2r0, sg10, sg20, ssc0),
                (sd1, rows1, t2r1, sg11, sg21, ssc1))

        def issue(g, b):
            sd, rows, t2r, sg1, sg2, _ = bufs[b]
            pltpu.sync_copy(sdi_hbm.at[cbase + g], sd)
            pltpu.async_copy(t1_hbm.at[sd.at[0]], rows, sg1)
            pltpu.async_copy(t2_hbm.at[sd.at[1]], t2r, sg2)

        def wait_gathers(b):
            sd, rows, t2r, sg1, sg2, _ = bufs[b]
            pltpu.make_async_copy(t1_hbm.at[sd.at[0]], rows, sg1).wait()
            pltpu.make_async_copy(t2_hbm.at[sd.at[1]], t2r, sg2).wait()

        def wait_scatter(b):
            sd, rows, _, _, _, ssc = bufs[b]
            pltpu.make_async_copy(rows, acc.at[sd.at[1]], ssc).wait()

        lrow = lbv[0, :]

        def compute(b):
            sd, rows, t2r, _, _, ssc = bufs[b]

            def pedge(e, c):
                asrc = rows[e, pl.ds(acol, 16)]
                adst = t2r[e, :]
                al = asrc + adst
                al = jnp.where(al > 0.0, al, al * 0.2)
                w = jnp.exp(al - lrow)
                wb[e, :] = w
                ev = jnp.full((16,), e, jnp.int32)
                for h in range(heads):
                    hv = jnp.full((16,), h, jnp.int32)
                    wbc = plsc.load_gather(wb, [ev, hv])
                    for t in range(csz // 16):
                        c0 = h * csz + t * 16
                        rows[e, pl.ds(c0, 16)] = rows[e, pl.ds(c0, 16)] * wbc
                rows[e, pl.ds(hc, 16)] = w
                return c
            lax.fori_loop(jnp.int32(0), jnp.int32(_K), pedge, jnp.int32(0))

            pltpu.async_copy(rows, acc.at[sd.at[1]], ssc, add=True)

        issue(jnp.int32(0), 0)

        def pair(gp, carry):
            for b in (0, 1):
                g = gp * jnp.int32(2) + jnp.int32(b)
                other = 1 - b
                pl.when(g >= 1)(lambda: wait_scatter(other))
                pl.when(g + 1 < _NCH)(lambda: issue(g + jnp.int32(1), other))
                wait_gathers(b)
                compute(b)
            return carry
        lax.fori_loop(jnp.int32(0), jnp.int32(_NCH // 2), pair, jnp.int32(0))
        wait_scatter(1)

        plsc.subcore_barrier()
        pltpu.sync_copy(acc.at[pl.ds(sid * _RPT, _RPT)],
                        out_hbm.at[cid, pl.ds(sid * _RPT, _RPT)])

    return ek


_edge1 = _make_edge_kernel(144, 128, 8, 16)
_edge2 = _make_edge_kernel(80, 64, 1, 64)


def kernel(x, edge_index, W1, att_src1, att_dst1, b1, W2, att_src2, att_dst2, b2):
    with jax.enable_x64(False):
        out = _run(x, edge_index, W1, att_src1, att_dst1, b1,
                   W2, att_src2, att_dst2, b2)
    return out.astype(jnp.float64)


def _run(x, edge_index, W1, att_src1, att_dst1, b1, W2, att_src2, att_dst2, b2):
    f32 = jnp.float32
    x = x.astype(f32)
    W1 = W1.astype(f32)
    W2 = W2.astype(f32)

    ei = edge_index.astype(jnp.int32)
    loop = jnp.arange(_N, dtype=jnp.int32)
    padv = jnp.full((_EPAD - (_E + _N),), _N, jnp.int32)
    src = jnp.concatenate([ei[0], loop, padv])
    dst = jnp.concatenate([ei[1], loop, padv])
    sdi = jnp.stack([src.reshape(-1, _K), dst.reshape(-1, _K)], axis=1)

    xp = jnp.pad(x, ((0, _NPAD - _N), (0, 0)))

    a1s = att_src1.reshape(8, 16).astype(f32)
    a1d = att_dst1.reshape(8, 16).astype(f32)
    eye8 = jnp.eye(8, dtype=f32)
    As1 = (a1s[:, :, None] * eye8[:, None, :]).reshape(128, 8)
    Ad1 = (a1d[:, :, None] * eye8[:, None, :]).reshape(128, 8)
    R = jnp.repeat(eye8, 16, axis=1)

    t1a, t2a = _prep1(xp, W1, As1, Ad1)
    Ls = jnp.max(t1a[:, 128:136], axis=0) + jnp.max(t2a[:, 0:8], axis=0)
    L1v = jnp.where(Ls > 0, Ls, 0.2 * Ls)
    lb1 = jnp.full((16, 16), 1e4, f32).at[0, 0:8].set(L1v)
    acc1 = _edge1(t1a, t2a, sdi, lb1, jnp.zeros((_NPAD, 144), f32))

    As2 = att_src2.reshape(64, 1).astype(f32)
    Ad2 = att_dst2.reshape(64, 1).astype(f32)
    t1b, t2b = _prep2(acc1, b1.reshape(1, 128).astype(f32), W2, As2, Ad2, R)
    Ls2 = jnp.max(t1b[:, 64]) + jnp.max(t2b[:, 0])
    L2v = jnp.where(Ls2 > 0, Ls2, 0.2 * Ls2)
    lb2 = jnp.full((16, 16), 1e4, f32).at[0, 0].set(L2v)
    acc2 = _edge2(t1b, t2b, sdi, lb2, jnp.zeros((_NPAD, 80), f32))

    o = _final(acc2, b2.reshape(1, 64).astype(f32))
    return o[:_N]


# speedup vs baseline: 1.0681x; 1.0681x over previous
import functools

import jax
import jax.numpy as jnp
from jax import lax
from jax.experimental import pallas as pl
from jax.experimental.pallas import tpu as pltpu
from jax.experimental.pallas import tpu_sc as plsc

_N = 10000
_E = 320000
_NPAD = 10240
_NW = 32
_K = 96
_EPW = 10368
_EPAD = _NW * _EPW
_NCH = _EPW // _K
_RPT = _NPAD // 16
_BLK = 1280
_G = _NPAD // _BLK


def _prep1_body(x_ref, w1_ref, as_ref, ad_ref, t1_ref, t2_ref):
    h = jnp.dot(x_ref[...], w1_ref[...], preferred_element_type=jnp.float32)
    asrc = lax.dot_general(h, as_ref[...], (((1,), (0,)), ((), ())))
    adst = lax.dot_general(h, ad_ref[...], (((1,), (0,)), ((), ())))
    z8 = jnp.zeros_like(asrc)
    t1_ref[...] = jnp.concatenate([h, asrc, z8], axis=1)
    t2_ref[...] = jnp.concatenate([adst, z8], axis=1)


def _prep1(xp, W1, As1, Ad1):
    return pl.pallas_call(
        _prep1_body,
        grid=(_G,),
        in_specs=[
            pl.BlockSpec((_BLK, 128), lambda i: (i, 0)),
            pl.BlockSpec((128, 128), lambda i: (0, 0)),
            pl.BlockSpec((128, 8), lambda i: (0, 0)),
            pl.BlockSpec((128, 8), lambda i: (0, 0)),
        ],
        out_specs=[
            pl.BlockSpec((_BLK, 144), lambda i: (i, 0)),
            pl.BlockSpec((_BLK, 16), lambda i: (i, 0)),
        ],
        out_shape=[
            jax.ShapeDtypeStruct((_NPAD, 144), jnp.float32),
            jax.ShapeDtypeStruct((_NPAD, 16), jnp.float32),
        ],
    )(xp, W1, As1, Ad1)


def _prep2_body(acc_ref, b1_ref, w2_ref, as2_ref, ad2_ref, r_ref, t1_ref, t2_ref):
    s = acc_ref[0] + acc_ref[1]
    den = s[:, 128:136]
    den = jnp.where(den != 0.0, den, 1.0)
    inv128 = lax.dot_general(1.0 / den, r_ref[...], (((1,), (0,)), ((), ())))
    h1 = s[:, 0:128] * inv128 + b1_ref[...]
    neg = jnp.exp(jnp.minimum(h1, 0.0)) - 1.0
    h1 = jnp.where(h1 > 0.0, h1, neg)
    h2 = jnp.dot(h1, w2_ref[...], preferred_element_type=jnp.float32)
    a2s = lax.dot_general(h2, as2_ref[...], (((1,), (0,)), ((), ())))
    a2d = lax.dot_general(h2, ad2_ref[...], (((1,), (0,)), ((), ())))
    z15 = jnp.zeros((h2.shape[0], 15), jnp.float32)
    t1_ref[...] = jnp.concatenate([h2, a2s, z15], axis=1)
    t2_ref[...] = jnp.concatenate([a2d, z15], axis=1)


def _prep2(acc1, b1, W2, As2, Ad2, R):
    return pl.pallas_call(
        _prep2_body,
        grid=(_G,),
        in_specs=[
            pl.BlockSpec((2, _BLK, 144), lambda i: (0, i, 0)),
            pl.BlockSpec((1, 128), lambda i: (0, 0)),
            pl.BlockSpec((128, 64), lambda i: (0, 0)),
            pl.BlockSpec((64, 1), lambda i: (0, 0)),
            pl.BlockSpec((64, 1), lambda i: (0, 0)),
            pl.BlockSpec((8, 128), lambda i: (0, 0)),
        ],
        out_specs=[
            pl.BlockSpec((_BLK, 80), lambda i: (i, 0)),
            pl.BlockSpec((_BLK, 16), lambda i: (i, 0)),
        ],
        out_shape=[
            jax.ShapeDtypeStruct((_NPAD, 80), jnp.float32),
            jax.ShapeDtypeStruct((_NPAD, 16), jnp.float32),
        ],
    )(acc1, b1, W2, As2, Ad2, R)


def _final_body(acc_ref, b2_ref, o_ref):
    s = acc_ref[0] + acc_ref[1]
    den = s[:, 64:65]
    den = jnp.where(den != 0.0, den, 1.0)
    o = s[:, 0:64] / den + b2_ref[...]
    m = jnp.max(o, axis=1, keepdims=True)
    z = o - m
    lse = jnp.log(jnp.sum(jnp.exp(z), axis=1, keepdims=True))
    o_ref[...] = z - lse


def _final(acc2, b2):
    return pl.pallas_call(
        _final_body,
        grid=(_G,),
        in_specs=[
            pl.BlockSpec((2, _BLK, 80), lambda i: (0, i, 0)),
            pl.BlockSpec((1, 64), lambda i: (0, 0)),
        ],
        out_specs=pl.BlockSpec((_BLK, 64), lambda i: (i, 0)),
        out_shape=jax.ShapeDtypeStruct((_NPAD, 64), jnp.float32),
    )(acc2, b2)


def _make_edge_kernel(pw, acol, heads, csz):
    hc = acol
    mesh = plsc.VectorSubcoreMesh(core_axis_name="c", subcore_axis_name="s")

    @functools.partial(
        pl.kernel,
        mesh=mesh,
        compiler_params=pltpu.CompilerParams(
            use_tc_tiling_on_sc=False, needs_layout_passes=False),
        out_type=jax.ShapeDtypeStruct((2, _NPAD, pw), jnp.float32),
        scratch_types=[
            pltpu.VMEM((2, _K), jnp.int32),
            pltpu.VMEM((2, _K), jnp.int32),
            pltpu.VMEM((_K, pw), jnp.float32),
            pltpu.VMEM((_K, pw), jnp.float32),
            pltpu.VMEM((_K, 16), jnp.float32),
            pltpu.VMEM((_K, 16), jnp.float32),
            pltpu.VMEM((_K, 16), jnp.float32),
            pltpu.VMEM((16, 16), jnp.float32),
            pltpu.VMEM_SHARED((_NPAD, pw), jnp.float32),
            pltpu.SemaphoreType.DMA,
            pltpu.SemaphoreType.DMA,
            pltpu.SemaphoreType.DMA,
            pltpu.SemaphoreType.DMA,
            pltpu.SemaphoreType.DMA,
            pltpu.SemaphoreType.DMA,
        ],
    )
    def ek(t1_hbm, t2_hbm, sdi_hbm, lb_hbm, zer_hbm, out_hbm,
           sd0, sd1, rows0, rows1, t2r0, t2r1, wb, lbv, acc,
           sg10, sg11, sg20, sg21, ssc0, ssc1):
        cid = lax.axis_index("c")
        sid = lax.axis_index("s")
        gwid = cid * 16 + sid

        pltpu.sync_copy(zer_hbm.at[pl.ds(sid * _RPT, _RPT)],
                        acc.at[pl.ds(sid * _RPT, _RPT)])
        pltpu.sync_copy(lb_hbm, lbv)

        def zw(i, carry):
            wb[i, :] = jnp.zeros((16,), jnp.float32)
            return carry
        lax.fori_loop(jnp.int32(0), jnp.int32(_K), zw, jnp.int32(0))
        plsc.subcore_barrier()

        cbase = gwid * jnp.int32(_NCH)
        bufs = ((sd0, rows0, t2r0, sg10, sg20, ssc0),
                (sd1, rows1, t2r1, sg11, sg21, ssc1))

        def issue(g, b):
            sd, rows, t2r, sg1, sg2, _ = bufs[b]
            pltpu.sync_copy(sdi_hbm.at[cbase + g], sd)
            pltpu.async_copy(t1_hbm.at[sd.at[0]], rows, sg1)
            pltpu.async_copy(t2_hbm.at[sd.at[1]], t2r, sg2)

        def wait_gathers(b):
            sd, rows, t2r, sg1, sg2, _ = bufs[b]
            pltpu.make_async_copy(t1_hbm.at[sd.at[0]], rows, sg1).wait()
            pltpu.make_async_copy(t2_hbm.at[sd.at[1]], t2r, sg2).wait()

        def wait_scatter(b):
            sd, rows, _, _, _, ssc = bufs[b]
            pltpu.make_async_copy(rows, acc.at[sd.at[1]], ssc).wait()

        def compute(b):
            sd, rows, t2r, _, _, ssc = bufs[b]

            def wgrp(q, c):
                lanes = lax.iota(jnp.int32, 16) + q * jnp.int32(16)
                for h in range(heads):
                    av = plsc.load_gather(
                        rows, [lanes, jnp.full((16,), acol + h, jnp.int32)])
                    hv = jnp.full((16,), h, jnp.int32)
                    dv = plsc.load_gather(t2r, [lanes, hv])
                    al = av + dv
                    al = jnp.where(al > 0.0, al, al * 0.2)
                    wv = jnp.exp(al - lbv[h, :])
                    plsc.store_scatter(wb, [lanes, hv], wv)
                return c
            lax.fori_loop(jnp.int32(0), jnp.int32(_K // 16), wgrp, jnp.int32(0))

            def pedge(e, c):
                ev = jnp.full((16,), e, jnp.int32)
                for h in range(heads):
                    wbc = plsc.load_gather(
                        wb, [ev, jnp.full((16,), h, jnp.int32)])
                    for t in range(csz // 16):
                        c0 = h * csz + t * 16
                        rows[e, pl.ds(c0, 16)] = rows[e, pl.ds(c0, 16)] * wbc
                rows[e, pl.ds(hc, 16)] = wb[e, :]
                return c
            lax.fori_loop(jnp.int32(0), jnp.int32(_K), pedge, jnp.int32(0))

            pltpu.async_copy(rows, acc.at[sd.at[1]], ssc, add=True)

        issue(jnp.int32(0), 0)

        def pair(gp, carry):
            for b in (0, 1):
                g = gp * jnp.int32(2) + jnp.int32(b)
                other = 1 - b
                pl.when(g >= 1)(lambda: wait_scatter(other))
                pl.when(g + 1 < _NCH)(lambda: issue(g + jnp.int32(1), other))
                wait_gathers(b)
                compute(b)
            return carry
        lax.fori_loop(jnp.int32(0), jnp.int32(_NCH // 2), pair, jnp.int32(0))
        wait_scatter(1)

        plsc.subcore_barrier()
        pltpu.sync_copy(acc.at[pl.ds(sid * _RPT, _RPT)],
                        out_hbm.at[cid, pl.ds(sid * _RPT, _RPT)])

    return ek


_edge1 = _make_edge_kernel(144, 128, 8, 16)
_edge2 = _make_edge_kernel(80, 64, 1, 64)


def kernel(x, edge_index, W1, att_src1, att_dst1, b1, W2, att_src2, att_dst2, b2):
    with jax.enable_x64(False):
        out = _run(x, edge_index, W1, att_src1, att_dst1, b1,
                   W2, att_src2, att_dst2, b2)
    return out.astype(jnp.float64)


def _run(x, edge_index, W1, att_src1, att_dst1, b1, W2, att_src2, att_dst2, b2):
    f32 = jnp.float32
    x = x.astype(f32)
    W1 = W1.astype(f32)
    W2 = W2.astype(f32)

    ei = edge_index.astype(jnp.int32)
    loop = jnp.arange(_N, dtype=jnp.int32)
    padv = jnp.full((_EPAD - (_E + _N),), _N, jnp.int32)
    src = jnp.concatenate([ei[0], loop, padv])
    dst = jnp.concatenate([ei[1], loop, padv])
    sdi = jnp.stack([src.reshape(-1, _K), dst.reshape(-1, _K)], axis=1)

    xp = jnp.pad(x, ((0, _NPAD - _N), (0, 0)))

    a1s = att_src1.reshape(8, 16).astype(f32)
    a1d = att_dst1.reshape(8, 16).astype(f32)
    eye8 = jnp.eye(8, dtype=f32)
    As1 = (a1s[:, :, None] * eye8[:, None, :]).reshape(128, 8)
    Ad1 = (a1d[:, :, None] * eye8[:, None, :]).reshape(128, 8)
    R = jnp.repeat(eye8, 16, axis=1)

    t1a, t2a = _prep1(xp, W1, As1, Ad1)
    Ls = jnp.max(t1a[:, 128:136], axis=0) + jnp.max(t2a[:, 0:8], axis=0)
    L1v = jnp.where(Ls > 0, Ls, 0.2 * Ls)
    lb1 = jnp.zeros((16, 16), f32).at[0:8, :].set(
        jnp.broadcast_to(L1v[:, None], (8, 16)))
    acc1 = _edge1(t1a, t2a, sdi, lb1, jnp.zeros((_NPAD, 144), f32))

    As2 = att_src2.reshape(64, 1).astype(f32)
    Ad2 = att_dst2.reshape(64, 1).astype(f32)
    t1b, t2b = _prep2(acc1, b1.reshape(1, 128).astype(f32), W2, As2, Ad2, R)
    Ls2 = jnp.max(t1b[:, 64]) + jnp.max(t2b[:, 0])
    L2v = jnp.where(Ls2 > 0, Ls2, 0.2 * Ls2)
    lb2 = jnp.zeros((16, 16), f32).at[0, :].set(L2v)
    acc2 = _edge2(t1b, t2b, sdi, lb2, jnp.zeros((_NPAD, 80), f32))

    o = _final(acc2, b2.reshape(1, 64).astype(f32))
    return o[:_N]


# speedup vs baseline: 1.0743x; 1.0059x over previous
import functools

import jax
import jax.numpy as jnp
from jax import lax
from jax.experimental import pallas as pl
from jax.experimental.pallas import tpu as pltpu
from jax.experimental.pallas import tpu_sc as plsc

_N = 10000
_E = 320000
_NPAD = 10240
_NW = 32
_K = 96
_EPW = 10368
_EPAD = _NW * _EPW
_NCH = _EPW // _K
_RPT = _NPAD // 16
_BLK = 1280
_G = _NPAD // _BLK


def _prep1_body(x_ref, w1_ref, as_ref, ad_ref, t1_ref, t2_ref):
    h = jnp.dot(x_ref[...], w1_ref[...], preferred_element_type=jnp.float32)
    asrc = lax.dot_general(h, as_ref[...], (((1,), (0,)), ((), ())))
    adst = lax.dot_general(h, ad_ref[...], (((1,), (0,)), ((), ())))
    z8 = jnp.zeros_like(asrc)
    t1_ref[...] = jnp.concatenate([h, asrc, z8], axis=1)
    t2_ref[...] = jnp.concatenate([adst, z8], axis=1)


def _prep1(xp, W1, As1, Ad1):
    return pl.pallas_call(
        _prep1_body,
        grid=(_G,),
        in_specs=[
            pl.BlockSpec((_BLK, 128), lambda i: (i, 0)),
            pl.BlockSpec((128, 128), lambda i: (0, 0)),
            pl.BlockSpec((128, 8), lambda i: (0, 0)),
            pl.BlockSpec((128, 8), lambda i: (0, 0)),
        ],
        out_specs=[
            pl.BlockSpec((_BLK, 144), lambda i: (i, 0)),
            pl.BlockSpec((_BLK, 16), lambda i: (i, 0)),
        ],
        out_shape=[
            jax.ShapeDtypeStruct((_NPAD, 144), jnp.float32),
            jax.ShapeDtypeStruct((_NPAD, 16), jnp.float32),
        ],
    )(xp, W1, As1, Ad1)


def _prep2_body(acc_ref, b1_ref, w2_ref, as2_ref, ad2_ref, r_ref, t1_ref, t2_ref):
    s = acc_ref[0] + acc_ref[1]
    den = s[:, 128:136]
    den = jnp.where(den != 0.0, den, 1.0)
    inv128 = lax.dot_general(1.0 / den, r_ref[...], (((1,), (0,)), ((), ())))
    h1 = s[:, 0:128] * inv128 + b1_ref[...]
    neg = jnp.exp(jnp.minimum(h1, 0.0)) - 1.0
    h1 = jnp.where(h1 > 0.0, h1, neg)
    h2 = jnp.dot(h1, w2_ref[...], preferred_element_type=jnp.float32)
    a2s = lax.dot_general(h2, as2_ref[...], (((1,), (0,)), ((), ())))
    a2d = lax.dot_general(h2, ad2_ref[...], (((1,), (0,)), ((), ())))
    z15 = jnp.zeros((h2.shape[0], 15), jnp.float32)
    t1_ref[...] = jnp.concatenate([h2, a2s, z15], axis=1)
    t2_ref[...] = jnp.concatenate([a2d, z15], axis=1)


def _prep2(acc1, b1, W2, As2, Ad2, R):
    return pl.pallas_call(
        _prep2_body,
        grid=(_G,),
        in_specs=[
            pl.BlockSpec((2, _BLK, 144), lambda i: (0, i, 0)),
            pl.BlockSpec((1, 128), lambda i: (0, 0)),
            pl.BlockSpec((128, 64), lambda i: (0, 0)),
            pl.BlockSpec((64, 1), lambda i: (0, 0)),
            pl.BlockSpec((64, 1), lambda i: (0, 0)),
            pl.BlockSpec((8, 128), lambda i: (0, 0)),
        ],
        out_specs=[
            pl.BlockSpec((_BLK, 80), lambda i: (i, 0)),
            pl.BlockSpec((_BLK, 16), lambda i: (i, 0)),
        ],
        out_shape=[
            jax.ShapeDtypeStruct((_NPAD, 80), jnp.float32),
            jax.ShapeDtypeStruct((_NPAD, 16), jnp.float32),
        ],
    )(acc1, b1, W2, As2, Ad2, R)


def _final_body(acc_ref, b2_ref, o_ref):
    s = acc_ref[0] + acc_ref[1]
    den = s[:, 64:65]
    den = jnp.where(den != 0.0, den, 1.0)
    o = s[:, 0:64] / den + b2_ref[...]
    m = jnp.max(o, axis=1, keepdims=True)
    z = o - m
    lse = jnp.log(jnp.sum(jnp.exp(z), axis=1, keepdims=True))
    o_ref[...] = z - lse


def _final(acc2, b2):
    return pl.pallas_call(
        _final_body,
        grid=(_G,),
        in_specs=[
            pl.BlockSpec((2, _BLK, 80), lambda i: (0, i, 0)),
            pl.BlockSpec((1, 64), lambda i: (0, 0)),
        ],
        out_specs=pl.BlockSpec((_BLK, 64), lambda i: (i, 0)),
        out_shape=jax.ShapeDtypeStruct((_NPAD, 64), jnp.float32),
    )(acc2, b2)


def _make_edge_kernel(pw, acol, heads, csz):
    hc = acol
    mesh = plsc.VectorSubcoreMesh(core_axis_name="c", subcore_axis_name="s")

    @functools.partial(
        pl.kernel,
        mesh=mesh,
        compiler_params=pltpu.CompilerParams(
            use_tc_tiling_on_sc=False, needs_layout_passes=False),
        out_type=jax.ShapeDtypeStruct((2, _NPAD, pw), jnp.float32),
        scratch_types=[
            pltpu.VMEM((2, _K), jnp.int32),
            pltpu.VMEM((2, _K), jnp.int32),
            pltpu.VMEM((_K, pw), jnp.float32),
            pltpu.VMEM((_K, pw), jnp.float32),
            pltpu.VMEM((_K, 16), jnp.float32),
            pltpu.VMEM((_K, 16), jnp.float32),
            pltpu.VMEM((_K, 16), jnp.float32),
            pltpu.VMEM((16, 16), jnp.float32),
            pltpu.VMEM_SHARED((_NPAD, pw), jnp.float32),
            pltpu.SemaphoreType.DMA,
            pltpu.SemaphoreType.DMA,
            pltpu.SemaphoreType.DMA,
            pltpu.SemaphoreType.DMA,
            pltpu.SemaphoreType.DMA,
            pltpu.SemaphoreType.DMA,
        ],
    )
    def ek(t1_hbm, t2_hbm, sdi_hbm, lb_hbm, zer_hbm, out_hbm,
           sd0, sd1, rows0, rows1, t2r0, t2r1, wb, lbv, acc,
           sg10, sg11, sg20, sg21, ssc0, ssc1):
        cid = lax.axis_index("c")
        sid = lax.axis_index("s")
        gwid = cid * 16 + sid

        pltpu.sync_copy(zer_hbm.at[pl.ds(sid * _RPT, _RPT)],
                        acc.at[pl.ds(sid * _RPT, _RPT)])
        pltpu.sync_copy(lb_hbm, lbv)

        def zw(i, carry):
            wb[i, :] = jnp.zeros((16,), jnp.float32)
            return carry
        lax.fori_loop(jnp.int32(0), jnp.int32(_K), zw, jnp.int32(0))
        plsc.subcore_barrier()

        cbase = gwid * jnp.int32(_NCH)
        bufs = ((sd0, rows0, t2r0, sg10, sg20, ssc0),
                (sd1, rows1, t2r1, sg11, sg21, ssc1))

        def issue(g, b):
            sd, rows, t2r, sg1, sg2, _ = bufs[b]
            pltpu.sync_copy(sdi_hbm.at[cbase + g], sd)
            pltpu.async_copy(t1_hbm.at[sd.at[0]], rows, sg1)
            pltpu.async_copy(t2_hbm.at[sd.at[1]], t2r, sg2)

        def wait_gathers(b):
            sd, rows, t2r, sg1, sg2, _ = bufs[b]
            pltpu.make_async_copy(t1_hbm.at[sd.at[0]], rows, sg1).wait()
            pltpu.make_async_copy(t2_hbm.at[sd.at[1]], t2r, sg2).wait()

        def wait_scatter(b):
            sd, rows, _, _, _, ssc = bufs[b]
            pltpu.make_async_copy(rows, acc.at[sd.at[1]], ssc).wait()

        def compute(b):
            sd, rows, t2r, _, _, ssc = bufs[b]

            def wgrp(q, c):
                lanes = lax.iota(jnp.int32, 16) + q * jnp.int32(16)
                for h in range(heads):
                    av = plsc.load_gather(
                        rows, [lanes, jnp.full((16,), acol + h, jnp.int32)])
                    hv = jnp.full((16,), h, jnp.int32)
                    dv = plsc.load_gather(t2r, [lanes, hv])
                    al = av + dv
                    al = jnp.where(al > 0.0, al, al * 0.2)
                    wv = jnp.exp(al - lbv[h, :])
                    plsc.store_scatter(wb, [lanes, hv], wv)
                return c
            lax.fori_loop(jnp.int32(0), jnp.int32(_K // 16), wgrp, jnp.int32(0))

            def pedge(ep, c):
                for u in range(2):
                    e = ep * jnp.int32(2) + jnp.int32(u)
                    ev = jnp.full((16,), e, jnp.int32)
                    for h in range(heads):
                        wbc = plsc.load_gather(
                            wb, [ev, jnp.full((16,), h, jnp.int32)])
                        for t in range(csz // 16):
                            c0 = h * csz + t * 16
                            rows[e, pl.ds(c0, 16)] = (
                                rows[e, pl.ds(c0, 16)] * wbc)
                    rows[e, pl.ds(hc, 16)] = wb[e, :]
                return c
            lax.fori_loop(jnp.int32(0), jnp.int32(_K // 2), pedge, jnp.int32(0))

            pltpu.async_copy(rows, acc.at[sd.at[1]], ssc, add=True)

        issue(jnp.int32(0), 0)

        def pair(gp, carry):
            for b in (0, 1):
                g = gp * jnp.int32(2) + jnp.int32(b)
                other = 1 - b
                pl.when(g >= 1)(lambda: wait_scatter(other))
                pl.when(g + 1 < _NCH)(lambda: issue(g + jnp.int32(1), other))
                wait_gathers(b)
                compute(b)
            return carry
        lax.fori_loop(jnp.int32(0), jnp.int32(_NCH // 2), pair, jnp.int32(0))
        wait_scatter(1)

        plsc.subcore_barrier()
        pltpu.sync_copy(acc.at[pl.ds(sid * _RPT, _RPT)],
                        out_hbm.at[cid, pl.ds(sid * _RPT, _RPT)])

    return ek


_edge1 = _make_edge_kernel(144, 128, 8, 16)
_edge2 = _make_edge_kernel(80, 64, 1, 64)


def kernel(x, edge_index, W1, att_src1, att_dst1, b1, W2, att_src2, att_dst2, b2):
    with jax.enable_x64(False):
        out = _run(x, edge_index, W1, att_src1, att_dst1, b1,
                   W2, att_src2, att_dst2, b2)
    return out.astype(jnp.float64)


def _run(x, edge_index, W1, att_src1, att_dst1, b1, W2, att_src2, att_dst2, b2):
    f32 = jnp.float32
    x = x.astype(f32)
    W1 = W1.astype(f32)
    W2 = W2.astype(f32)

    ei = edge_index.astype(jnp.int32)
    loop = jnp.arange(_N, dtype=jnp.int32)
    padv = jnp.full((_EPAD - (_E + _N),), _N, jnp.int32)
    src = jnp.concatenate([ei[0], loop, padv])
    dst = jnp.concatenate([ei[1], loop, padv])
    sdi = jnp.stack([src.reshape(-1, _K), dst.reshape(-1, _K)], axis=1)

    xp = jnp.pad(x, ((0, _NPAD - _N), (0, 0)))

    a1s = att_src1.reshape(8, 16).astype(f32)
    a1d = att_dst1.reshape(8, 16).astype(f32)
    eye8 = jnp.eye(8, dtype=f32)
    As1 = (a1s[:, :, None] * eye8[:, None, :]).reshape(128, 8)
    Ad1 = (a1d[:, :, None] * eye8[:, None, :]).reshape(128, 8)
    R = jnp.repeat(eye8, 16, axis=1)

    t1a, t2a = _prep1(xp, W1, As1, Ad1)
    Ls = jnp.max(t1a[:, 128:136], axis=0) + jnp.max(t2a[:, 0:8], axis=0)
    L1v = jnp.where(Ls > 0, Ls, 0.2 * Ls)
    lb1 = jnp.zeros((16, 16), f32).at[0:8, :].set(
        jnp.broadcast_to(L1v[:, None], (8, 16)))
    acc1 = _edge1(t1a, t2a, sdi, lb1, jnp.zeros((_NPAD, 144), f32))

    As2 = att_src2.reshape(64, 1).astype(f32)
    Ad2 = att_dst2.reshape(64, 1).astype(f32)
    t1b, t2b = _prep2(acc1, b1.reshape(1, 128).astype(f32), W2, As2, Ad2, R)
    Ls2 = jnp.max(t1b[:, 64]) + jnp.max(t2b[:, 0])
    L2v = jnp.where(Ls2 > 0, Ls2, 0.2 * Ls2)
    lb2 = jnp.zeros((16, 16), f32).at[0, :].set(L2v)
    acc2 = _edge2(t1b, t2b, sdi, lb2, jnp.zeros((_NPAD, 80), f32))

    o = _final(acc2, b2.reshape(1, 64).astype(f32))
    return o[:_N]
